# Initial kernel scaffold; baseline (speedup 1.0000x reference)
import jax
import jax.numpy as jnp
from jax.experimental import pallas as pl


def kernel(s, y, Z):
    raise NotImplementedError("write your pallas kernel here")


# speedup vs baseline: 8.0437x; 8.0437x over previous
import functools

import jax
import jax.numpy as jnp
from jax import lax
from jax.experimental import pallas as pl
from jax.experimental.pallas import tpu as pltpu
from jax.experimental.pallas import tpu_sc as plsc

_B = 64
_V = 100000
_NS = 5
_K = 5
_EPS = 1.0

_NC = 2
_NSUB = 16
_NW = _NC * _NSUB
_L = 16
_ROWS_PER_W = _B // _NW
_VC = 4000
_NCHUNK = _V // _VC
_NVREG = _VC // _L
_KK = _K + 1

_NEG = float("-inf")


def _iota16():
    return lax.iota(jnp.int32, _L)


def _vec_i32(x):
    return _iota16() * 0 + x


def _kth_of_lists(ms):
    need = jnp.int32(_KK)
    ans = jnp.float32(0.0)
    t = jnp.float32(jnp.inf)
    for _ in range(_KK):
        cur = jnp.float32(_NEG)
        for r in range(_KK):
            cur = jnp.maximum(cur, jnp.max(jnp.where(ms[r] < t, ms[r], _NEG)))
        c = jnp.int32(0)
        for r in range(_KK):
            c = c + jnp.sum((ms[r] == cur).astype(jnp.int32))
        hit = jnp.logical_and(need > 0, need <= c)
        ans = jnp.where(hit, cur, ans)
        need = need - c
        t = cur
    return ans


def _sc_partials(s_flat, y, z_flat):
    mesh = plsc.VectorSubcoreMesh(core_axis_name="c", subcore_axis_name="s")

    @functools.partial(
        pl.kernel,
        out_type=jax.ShapeDtypeStruct((_NW * _L,), jnp.float32),
        mesh=mesh,
        compiler_params=pltpu.CompilerParams(needs_layout_passes=False),
        scratch_types=[
            pltpu.VMEM((_VC * _NS,), jnp.float32),
            pltpu.VMEM((_VC,), jnp.float32),
            pltpu.VMEM((_B,), jnp.int32),
            pltpu.VMEM((_L,), jnp.float32),
        ],
    )
    def body(s_hbm, y_hbm, z_hbm, out_hbm, zbuf, sbuf, ybuf, obuf):
        wid = lax.axis_index("s") * _NC + lax.axis_index("c")
        iota = _iota16()

        pltpu.sync_copy(y_hbm, ybuf)
        b_first = wid * _ROWS_PER_W
        bvec = jnp.minimum(_vec_i32(b_first) + iota, _B - 1)
        yg = plsc.load_gather(ybuf, [bvec])

        out_lanes = jnp.full((_L,), 0.0, jnp.float32)

        for rb in range(_ROWS_PER_W):
            b = b_first + rb
            y_b = jnp.max(jnp.where(iota == rb, yg, 0))

            def chunk_body(c, carry, b=b, y_b=y_b):
                ms_flat, corr = carry
                zoff = pl.multiple_of((b * _V + c * _VC) * _NS, 8)
                soff = pl.multiple_of(b * _V + c * _VC, 8)
                pltpu.sync_copy(z_hbm.at[pl.ds(zoff, _VC * _NS)], zbuf)
                pltpu.sync_copy(s_hbm.at[pl.ds(soff, _VC)], sbuf)

                off = y_b - c * _VC
                in_rng = jnp.logical_and(off >= 0, off < _VC)
                offc = jnp.clip(off, 0, _VC - 1)
                cv = plsc.load_gather(sbuf, [_vec_i32(offc)])
                corr = jnp.where(in_rng, jnp.max(cv), corr)

                new = []
                for j in range(_NS):
                    def inner(i, ms, j=j):
                        sv = sbuf[pl.ds(i * _L, _L)]
                        zidx = _vec_i32(i * (_L * _NS) + j) + iota * _NS
                        zv = plsc.load_gather(zbuf, [zidx])
                        p = sv + _EPS * zv
                        out = []
                        for r in range(_KK):
                            hi = jnp.maximum(ms[r], p)
                            p = jnp.minimum(ms[r], p)
                            out.append(hi)
                        return tuple(out)

                    ms = lax.fori_loop(
                        0, _NVREG, inner,
                        tuple(ms_flat[j * _KK + r] for r in range(_KK)))
                    new.extend(ms)
                return tuple(new), corr

            init_ms = tuple(jnp.full((_L,), _NEG, jnp.float32)
                            for _ in range(_NS * _KK))
            fin, corr = lax.fori_loop(0, _NCHUNK, chunk_body,
                                      (init_ms, jnp.float32(0.0)))

            acc = jnp.float32(0.0)
            for j in range(_NS):
                acc = acc + _kth_of_lists([fin[j * _KK + r] for r in range(_KK)])
            skp1 = acc * jnp.float32(1.0 / _NS)

            num = jnp.maximum(jnp.float32(0.0), jnp.float32(1.0) + skp1 - corr)
            out_lanes = jnp.where(iota == rb, num, out_lanes)

        obuf[...] = out_lanes
        pltpu.sync_copy(obuf, out_hbm.at[pl.ds(wid * _L, _L)])

    return body(s_flat, y, z_flat)


def _tc_mean(parts4x128):
    def fin(x_ref, o_ref):
        o_ref[0, 0] = jnp.sum(x_ref[...]) * jnp.float32(1.0 / _B)

    return pl.pallas_call(
        fin,
        out_shape=jax.ShapeDtypeStruct((1, 1), jnp.float32),
        out_specs=pl.BlockSpec(memory_space=pltpu.SMEM),
    )(parts4x128)


def kernel(s, y, Z):
    parts = _sc_partials(s.reshape(_B * _V), y.astype(jnp.int32),
                         Z.reshape(_B * _V * _NS))
    return _tc_mean(parts.reshape(4, 128))[0, 0]


# speedup vs baseline: 8.4139x; 1.0460x over previous
import functools

import jax
import jax.numpy as jnp
from jax import lax
from jax.experimental import pallas as pl
from jax.experimental.pallas import tpu as pltpu
from jax.experimental.pallas import tpu_sc as plsc

_B = 64
_V = 100000
_NS = 5
_K = 5
_EPS = 1.0

_NC = 2
_NSUB = 16
_NW = _NC * _NSUB
_L = 16
_ROWS_PER_W = _B // _NW
_VC = 2000
_NCHUNK = _V // _VC
_NG = _NCHUNK // 2
_NVREG = _VC // _L
_KK = _K + 1

_NEG = float("-inf")


def _iota16():
    return lax.iota(jnp.int32, _L)


def _vec_i32(x):
    return _iota16() * 0 + x


def _kth_of_lists(ms):
    need = jnp.int32(_KK)
    ans = jnp.float32(0.0)
    t = jnp.float32(jnp.inf)
    for _ in range(_KK):
        cur = jnp.float32(_NEG)
        for r in range(_KK):
            cur = jnp.maximum(cur, jnp.max(jnp.where(ms[r] < t, ms[r], _NEG)))
        c = jnp.int32(0)
        for r in range(_KK):
            c = c + jnp.sum((ms[r] == cur).astype(jnp.int32))
        hit = jnp.logical_and(need > 0, need <= c)
        ans = jnp.where(hit, cur, ans)
        need = need - c
        t = cur
    return ans


def _sc_partials(s_flat, y, z_flat):
    mesh = plsc.VectorSubcoreMesh(core_axis_name="c", subcore_axis_name="s")

    @functools.partial(
        pl.kernel,
        out_type=jax.ShapeDtypeStruct((_NW * _L,), jnp.float32),
        mesh=mesh,
        compiler_params=pltpu.CompilerParams(needs_layout_passes=False),
        scratch_types=[
            pltpu.VMEM((_VC * _NS,), jnp.float32),
            pltpu.VMEM((_VC * _NS,), jnp.float32),
            pltpu.VMEM((_VC,), jnp.float32),
            pltpu.VMEM((_VC,), jnp.float32),
            pltpu.VMEM((_B,), jnp.int32),
            pltpu.VMEM((_L,), jnp.float32),
            pltpu.SemaphoreType.DMA,
            pltpu.SemaphoreType.DMA,
        ],
    )
    def body(s_hbm, y_hbm, z_hbm, out_hbm,
             zbuf0, zbuf1, sbuf0, sbuf1, ybuf, obuf, sem0, sem1):
        wid = lax.axis_index("s") * _NC + lax.axis_index("c")
        iota = _iota16()
        bufs = ((zbuf0, sbuf0, sem0), (zbuf1, sbuf1, sem1))

        pltpu.sync_copy(y_hbm, ybuf)
        b_first = wid * _ROWS_PER_W
        bvec = jnp.minimum(_vec_i32(b_first) + iota, _B - 1)
        yg = plsc.load_gather(ybuf, [bvec])

        def start(b, c, zb, sb, sem):
            zoff = pl.multiple_of((b * _V + c * _VC) * _NS, 8)
            soff = pl.multiple_of(b * _V + c * _VC, 8)
            pltpu.async_copy(z_hbm.at[pl.ds(zoff, _VC * _NS)], zb, sem)
            pltpu.async_copy(s_hbm.at[pl.ds(soff, _VC)], sb, sem)

        def wait(zb, sb, sem):
            pltpu.make_async_copy(z_hbm.at[pl.ds(0, _VC * _NS)], zb, sem).wait()
            pltpu.make_async_copy(s_hbm.at[pl.ds(0, _VC)], sb, sem).wait()

        out_lanes = jnp.full((_L,), 0.0, jnp.float32)

        for rb in range(_ROWS_PER_W):
            b = b_first + rb
            y_b = jnp.max(jnp.where(iota == rb, yg, 0))

            for half in range(2):
                start(b, jnp.int32(half), *bufs[half])

            def g_body(g, carry, b=b, y_b=y_b):
                ms, corr = carry
                for half in range(2):
                    zb, sb, sem = bufs[half]
                    c = g * 2 + half
                    wait(zb, sb, sem)

                    off = y_b - c * _VC
                    in_rng = jnp.logical_and(off >= 0, off < _VC)
                    offc = jnp.clip(off, 0, _VC - 1)
                    cv = plsc.load_gather(sb, [_vec_i32(offc)])
                    corr = jnp.where(in_rng, jnp.max(cv), corr)

                    def inner(i, ms, zb=zb, sb=sb):
                        sv = sb[pl.ds(i * _L, _L)]
                        lane5 = iota * _NS
                        out = []
                        for j in range(_NS):
                            zidx = _vec_i32(i * (_L * _NS) + j) + lane5
                            zv = plsc.load_gather(zb, [zidx])
                            p = sv + _EPS * zv
                            for r in range(_KK):
                                hi = jnp.maximum(ms[j * _KK + r], p)
                                p = jnp.minimum(ms[j * _KK + r], p)
                                out.append(hi)
                        return tuple(out)

                    ms = plsc.parallel_loop(0, _NVREG, 1, unroll=2,
                                            carry=ms)(inner)

                    @pl.when(g < _NG - 1)
                    def _(b=b, c=c, half=half):
                        start(b, c + 2, *bufs[half])

                return ms, corr

            init_ms = tuple(jnp.full((_L,), _NEG, jnp.float32)
                            for _ in range(_NS * _KK))
            fin, corr = lax.fori_loop(0, _NG, g_body,
                                      (init_ms, jnp.float32(0.0)))

            acc = jnp.float32(0.0)
            for j in range(_NS):
                acc = acc + _kth_of_lists([fin[j * _KK + r] for r in range(_KK)])
            skp1 = acc * jnp.float32(1.0 / _NS)

            num = jnp.maximum(jnp.float32(0.0), jnp.float32(1.0) + skp1 - corr)
            out_lanes = jnp.where(iota == rb, num, out_lanes)

        obuf[...] = out_lanes
        pltpu.sync_copy(obuf, out_hbm.at[pl.ds(wid * _L, _L)])

    return body(s_flat, y, z_flat)


def _tc_mean(parts4x128):
    def fin(x_ref, o_ref):
        o_ref[0, 0] = jnp.sum(x_ref[...]) * jnp.float32(1.0 / _B)

    return pl.pallas_call(
        fin,
        out_shape=jax.ShapeDtypeStruct((1, 1), jnp.float32),
        out_specs=pl.BlockSpec(memory_space=pltpu.SMEM),
    )(parts4x128)


def kernel(s, y, Z):
    parts = _sc_partials(s.reshape(_B * _V), y.astype(jnp.int32),
                         Z.reshape(_B * _V * _NS))
    return _tc_mean(parts.reshape(4, 128))[0, 0]


# speedup vs baseline: 75.4856x; 8.9715x over previous
import functools

import jax
import jax.numpy as jnp
from jax import lax
from jax.experimental import pallas as pl
from jax.experimental.pallas import tpu as pltpu
from jax.experimental.pallas import tpu_sc as plsc

_B = 64
_V = 100000
_NS = 5
_K = 5
_EPS = 1.0

_NC = 2
_NSUB = 16
_NW = _NC * _NSUB
_L = 16
_ROWS_PER_W = _B // _NW
_VC = 2000
_NCHUNK = _V // _VC
_NG = _NCHUNK // 2
_NVREG = _VC // _L
_KK = _K + 1

_NEG = float("-inf")


def _iota16():
    return lax.iota(jnp.int32, _L)


def _vec_i32(x):
    return _iota16() * 0 + x


def _kth_of_lists(ms):
    need = jnp.int32(_KK)
    ans = jnp.float32(0.0)
    t = jnp.float32(jnp.inf)
    for _ in range(_KK):
        cur = jnp.float32(_NEG)
        for r in range(_KK):
            cur = jnp.maximum(cur, jnp.max(jnp.where(ms[r] < t, ms[r], _NEG)))
        c = jnp.int32(0)
        for r in range(_KK):
            c = c + jnp.sum((ms[r] == cur).astype(jnp.int32))
        hit = jnp.logical_and(need > 0, need <= c)
        ans = jnp.where(hit, cur, ans)
        need = need - c
        t = cur
    return ans


def _sc_partials(s_flat, y, z_flat):
    mesh = plsc.VectorSubcoreMesh(core_axis_name="c", subcore_axis_name="s")

    @functools.partial(
        pl.kernel,
        out_type=jax.ShapeDtypeStruct((_NW * _L,), jnp.float32),
        mesh=mesh,
        compiler_params=pltpu.CompilerParams(needs_layout_passes=False),
        scratch_types=[
            pltpu.VMEM((_VC * _NS,), jnp.float32),
            pltpu.VMEM((_VC * _NS,), jnp.float32),
            pltpu.VMEM((_VC,), jnp.float32),
            pltpu.VMEM((_VC,), jnp.float32),
            pltpu.VMEM((_B,), jnp.int32),
            pltpu.VMEM((_L,), jnp.float32),
            pltpu.SemaphoreType.DMA,
            pltpu.SemaphoreType.DMA,
        ],
    )
    def body(s_hbm, y_hbm, z_hbm, out_hbm,
             zbuf0, zbuf1, sbuf0, sbuf1, ybuf, obuf, sem0, sem1):
        wid = lax.axis_index("s") * _NC + lax.axis_index("c")
        iota = _iota16()
        bufs = ((zbuf0, sbuf0, sem0), (zbuf1, sbuf1, sem1))

        pltpu.sync_copy(y_hbm, ybuf)
        b_first = wid * _ROWS_PER_W
        bvec = jnp.minimum(_vec_i32(b_first) + iota, _B - 1)
        yg = plsc.load_gather(ybuf, [bvec])

        def start(b, c, zb, sb, sem):
            for j in range(_NS):
                zoff = pl.multiple_of((j * _B + b) * _V + c * _VC, 8)
                pltpu.async_copy(z_hbm.at[pl.ds(zoff, _VC)],
                                 zb.at[pl.ds(j * _VC, _VC)], sem)
            soff = pl.multiple_of(b * _V + c * _VC, 8)
            pltpu.async_copy(s_hbm.at[pl.ds(soff, _VC)], sb, sem)

        def wait(zb, sb, sem):
            pltpu.make_async_copy(z_hbm.at[pl.ds(0, _VC * _NS)], zb, sem).wait()
            pltpu.make_async_copy(s_hbm.at[pl.ds(0, _VC)], sb, sem).wait()

        out_lanes = jnp.full((_L,), 0.0, jnp.float32)

        for rb in range(_ROWS_PER_W):
            b = b_first + rb
            y_b = jnp.max(jnp.where(iota == rb, yg, 0))

            for half in range(2):
                start(b, jnp.int32(half), *bufs[half])

            def g_body(g, carry, b=b, y_b=y_b):
                ms, corr = carry
                for half in range(2):
                    zb, sb, sem = bufs[half]
                    c = g * 2 + half
                    wait(zb, sb, sem)

                    off = y_b - c * _VC
                    in_rng = jnp.logical_and(off >= 0, off < _VC)
                    offc = jnp.clip(off, 0, _VC - 1)
                    cv = plsc.load_gather(sb, [_vec_i32(offc)])
                    corr = jnp.where(in_rng, jnp.max(cv), corr)

                    def inner(i, ms, zb=zb, sb=sb):
                        sv = sb[pl.ds(i * _L, _L)]
                        out = []
                        for j in range(_NS):
                            zv = zb[pl.ds(j * _VC + i * _L, _L)]
                            p = sv + _EPS * zv
                            for r in range(_KK):
                                hi = jnp.maximum(ms[j * _KK + r], p)
                                p = jnp.minimum(ms[j * _KK + r], p)
                                out.append(hi)
                        return tuple(out)

                    ms = plsc.parallel_loop(0, _NVREG, 1, unroll=2,
                                            carry=ms)(inner)

                    @pl.when(g < _NG - 1)
                    def _(b=b, c=c, half=half):
                        start(b, c + 2, *bufs[half])

                return ms, corr

            init_ms = tuple(jnp.full((_L,), _NEG, jnp.float32)
                            for _ in range(_NS * _KK))
            fin, corr = lax.fori_loop(0, _NG, g_body,
                                      (init_ms, jnp.float32(0.0)))

            acc = jnp.float32(0.0)
            for j in range(_NS):
                acc = acc + _kth_of_lists([fin[j * _KK + r] for r in range(_KK)])
            skp1 = acc * jnp.float32(1.0 / _NS)

            num = jnp.maximum(jnp.float32(0.0), jnp.float32(1.0) + skp1 - corr)
            out_lanes = jnp.where(iota == rb, num, out_lanes)

        obuf[...] = out_lanes
        pltpu.sync_copy(obuf, out_hbm.at[pl.ds(wid * _L, _L)])

    return body(s_flat, y, z_flat)


def _tc_mean(parts4x128):
    def fin(x_ref, o_ref):
        o_ref[0, 0] = jnp.sum(x_ref[...]) * jnp.float32(1.0 / _B)

    return pl.pallas_call(
        fin,
        out_shape=jax.ShapeDtypeStruct((1, 1), jnp.float32),
        out_specs=pl.BlockSpec(memory_space=pltpu.SMEM),
    )(parts4x128)


def kernel(s, y, Z):
    zt = jnp.transpose(Z, (2, 0, 1)).reshape(_NS * _B * _V)
    parts = _sc_partials(s.reshape(_B * _V), y.astype(jnp.int32), zt)
    return _tc_mean(parts.reshape(4, 128))[0, 0]


# speedup vs baseline: 140.9598x; 1.8674x over previous
import functools

import jax
import jax.numpy as jnp
from jax import lax
from jax.experimental import pallas as pl
from jax.experimental.pallas import tpu as pltpu
from jax.experimental.pallas import tpu_sc as plsc

_B = 64
_V = 100000
_NS = 5
_K = 5
_EPS = 1.0

_NC = 2
_NSUB = 16
_NW = _NC * _NSUB
_L = 16
_KK = _K + 1

_NBLK = 8
_NQ = 4
_RPB = _B // _NBLK
_VC = 768
_CPQ = 32
_VMAIN = _NQ * _CPQ * _VC
_NEXTRA = 2
_TAILV = _VMAIN + _NEXTRA * _VC
_TAILN = _V - _TAILV
_NIT = _VC // _L
_NST = _NS * _RPB * _KK * _L

_NEG = float("-inf")


def _iota16():
    return lax.iota(jnp.int32, _L)


def _vec_i32(x):
    return _iota16() * 0 + x


def _sc_partials(s2d, y, zt3, s_tail, z_tail):
    mesh = plsc.VectorSubcoreMesh(core_axis_name="c", subcore_axis_name="s")

    @functools.partial(
        pl.kernel,
        out_type=(
            jax.ShapeDtypeStruct((_NW * _NST,), jnp.float32),
            jax.ShapeDtypeStruct((_NW * _L,), jnp.float32),
        ),
        mesh=mesh,
        compiler_params=pltpu.CompilerParams(needs_layout_passes=False),
        scratch_types=[
            pltpu.VMEM((_NS, _RPB, _VC), jnp.float32),
            pltpu.VMEM((_NS, _RPB, _VC), jnp.float32),
            pltpu.VMEM((_RPB, _VC), jnp.float32),
            pltpu.VMEM((_RPB, _VC), jnp.float32),
            pltpu.VMEM((_NST,), jnp.float32),
            pltpu.VMEM((_B,), jnp.int32),
            pltpu.VMEM((_L,), jnp.float32),
            pltpu.SemaphoreType.DMA,
            pltpu.SemaphoreType.DMA,
        ],
    )
    def body(s_hbm, y_hbm, z_hbm, st_hbm, zt_hbm, cand_hbm, corr_hbm,
             zbuf0, zbuf1, sbuf0, sbuf1, st, ybuf, cbuf, sem0, sem1):
        wid = lax.axis_index("s") * _NC + lax.axis_index("c")
        iota = _iota16()
        bufs = ((zbuf0, sbuf0, sem0), (zbuf1, sbuf1, sem1))

        blk = wid // _NQ
        q = wid % _NQ
        row0 = pl.multiple_of(blk * _RPB, 8)
        c_lo = q * _CPQ

        pltpu.sync_copy(y_hbm, ybuf)
        yv8 = plsc.load_gather(ybuf, [row0 + (iota & (_RPB - 1))])
        lane_ok = iota < _RPB

        vneg = jnp.full((_L,), _NEG, jnp.float32)

        def init_st(t, _):
            st[pl.ds(t * _L, _L)] = vneg
            return 0

        lax.fori_loop(0, _NST // _L, init_st, 0)

        def copies(c, zb, sb, zsrc, ssrc):
            v0 = pl.multiple_of(c * _VC, 128)
            out = []
            for j in range(_NS):
                out.append((zsrc.at[j, pl.ds(row0, _RPB), pl.ds(v0, _VC)],
                            zb.at[j]))
            out.append((ssrc.at[pl.ds(row0, _RPB), pl.ds(v0, _VC)], sb))
            return out

        def start(c, zb, sb, sem, zsrc=None, ssrc=None):
            zsrc = z_hbm if zsrc is None else zsrc
            ssrc = s_hbm if ssrc is None else ssrc
            for src, dst in copies(c, zb, sb, zsrc, ssrc):
                pltpu.async_copy(src, dst, sem)

        def wait(c, zb, sb, sem, zsrc=None, ssrc=None):
            zsrc = z_hbm if zsrc is None else zsrc
            ssrc = s_hbm if ssrc is None else ssrc
            for src, dst in copies(c, zb, sb, zsrc, ssrc):
                pltpu.make_async_copy(src, dst, sem).wait()

        def corr_update(sb, base, corr):
            off = yv8 - base
            inr = jnp.logical_and(off >= 0, off < _VC)
            offc = jnp.clip(off, 0, _VC - 1)
            g = plsc.load_gather(sb, [iota & (_RPB - 1), offc])
            take = jnp.logical_and(inr, lane_ok)
            return corr + jnp.where(take, g, 0.0)

        def compute_chunk(zb, sb):
            def rbody(i, _, zb=zb, sb=sb):
                sbase = i * (_KK * _NS * _L)
                ms = []
                for j in range(_NS):
                    for r in range(_KK):
                        ms.append(st[pl.ds(sbase + (j * _KK + r) * _L, _L)])

                def inner(v, ms, zb=zb, sb=sb, i=i):
                    sv = sb[i, pl.ds(v * _L, _L)]
                    out = []
                    for j in range(_NS):
                        zv = zb[j, i, pl.ds(v * _L, _L)]
                        p = sv + _EPS * zv
                        for r in range(_KK):
                            hi = jnp.maximum(ms[j * _KK + r], p)
                            p = jnp.minimum(ms[j * _KK + r], p)
                            out.append(hi)
                    return tuple(out)

                ms = lax.fori_loop(0, _NIT, inner, tuple(ms))
                for j in range(_NS):
                    for r in range(_KK):
                        st[pl.ds(sbase + (j * _KK + r) * _L, _L)] = \
                            ms[j * _KK + r]
                return 0

            lax.fori_loop(0, _RPB, rbody, 0)

        for half in range(2):
            start(c_lo + half, bufs[half][0], bufs[half][1], bufs[half][2])

        def g_body(g, corr):
            for half in range(2):
                zb, sb, sem = bufs[half]
                c = c_lo + g * 2 + half
                wait(c, zb, sb, sem)
                corr = corr_update(sb, c * _VC, corr)
                compute_chunk(zb, sb)

                @pl.when(g < _CPQ // 2 - 1)
                def _(c=c, zb=zb, sb=sb, sem=sem):
                    start(c + 2, zb, sb, sem)

            return corr

        corr = lax.fori_loop(0, _CPQ // 2, g_body,
                             jnp.full((_L,), 0.0, jnp.float32))

        @pl.when(q == _NQ - 1)
        def _():
            c2 = corr
            for e in range(_NEXTRA):
                zb, sb, sem = bufs[e]
                ce = _VMAIN // _VC + e
                start(ce, zb, sb, sem)
            for e in range(_NEXTRA):
                zb, sb, sem = bufs[e]
                ce = _VMAIN // _VC + e
                wait(ce, zb, sb, sem)
                c2 = corr_update(sb, ce * _VC, c2)
                compute_chunk(zb, sb)
            zb, sb, sem = bufs[0]
            start(0, zb, sb, sem, zsrc=zt_hbm, ssrc=st_hbm)
            wait(0, zb, sb, sem, zsrc=zt_hbm, ssrc=st_hbm)
            c2 = corr_update(sb, _TAILV, c2)
            compute_chunk(zb, sb)
            cbuf[...] = c2
            pltpu.sync_copy(cbuf, corr_hbm.at[pl.ds(wid * _L, _L)])

        @pl.when(q != _NQ - 1)
        def _():
            cbuf[...] = corr
            pltpu.sync_copy(cbuf, corr_hbm.at[pl.ds(wid * _L, _L)])

        pltpu.sync_copy(st, cand_hbm.at[pl.ds(wid * _NST, _NST)])

    return body(s2d, y, zt3, s_tail, z_tail)


def _tc_loss(cand2d, corr2d):
    def fin(x_ref, c_ref, o_ref):
        x = x_ref[...]
        need = jnp.full((_NS * _B, 1), _KK, jnp.int32)
        ans = jnp.zeros((_NS * _B, 1), jnp.float32)
        t = jnp.full((_NS * _B, 1), jnp.inf, jnp.float32)
        for _ in range(_KK):
            masked = jnp.where(x < t, x, _NEG)
            m = jnp.max(masked, axis=1, keepdims=True)
            c = jnp.sum((x == m).astype(jnp.int32), axis=1, keepdims=True)
            hit = jnp.logical_and(need > 0, need <= c)
            ans = jnp.where(hit, m, ans)
            need = need - c
            t = m
        skp1 = jnp.zeros((_B, 1), jnp.float32)
        for j in range(_NS):
            skp1 = skp1 + ans[j * _B:(j + 1) * _B, :]
        skp1 = skp1 * jnp.float32(1.0 / _NS)
        corr = jnp.sum(c_ref[...], axis=1, keepdims=True)
        num = jnp.maximum(jnp.float32(1.0) + skp1 - corr, 0.0)
        o_ref[0, 0] = jnp.sum(num) * jnp.float32(1.0 / _B)

    return pl.pallas_call(
        fin,
        out_shape=jax.ShapeDtypeStruct((1, 1), jnp.float32),
        out_specs=pl.BlockSpec(memory_space=pltpu.SMEM),
    )(cand2d, corr2d)


def kernel(s, y, Z):
    zt3 = jnp.transpose(Z, (2, 0, 1))
    pad = _VC - _TAILN
    s_tail = jnp.pad(s[:, _TAILV:], ((0, 0), (0, pad)))
    z_tail = jnp.pad(zt3[:, :, _TAILV:], ((0, 0), (0, 0), (0, pad)),
                     constant_values=_NEG)
    cand, corr = _sc_partials(s, y.astype(jnp.int32), zt3, s_tail, z_tail)
    cand2d = cand.reshape(_NBLK, _NQ, _RPB, _NS, _KK * _L)
    cand2d = cand2d.transpose(3, 0, 2, 1, 4).reshape(_NS * _B, _NQ * _KK * _L)
    corr2d = (corr.reshape(_NBLK, _NQ, _L)[:, :, :_RPB]
              .transpose(0, 2, 1).reshape(_B, _NQ))
    return _tc_loss(cand2d, corr2d)[0, 0]
